# baseline (speedup 1.0000x reference)
import functools

import jax
import jax.numpy as jnp
from jax import lax
from jax.experimental import pallas as pl
from jax.experimental.pallas import tpu as pltpu
from jax.experimental.pallas import tpu_sc as plsc

D_MODEL = 1024
HIDDEN = 4096
N_EXP = 8
K_TOP = 2
N_TOK = 2048

TB = 128
TH = 512
NB = (N_TOK * K_TOP) // TB + N_EXP
R_PAD = NB * TB
NH = HIDDEN // TH


def _router_body(x_ref, wr_ref, br_ref, idx_ref, w_ref):
    logits = jnp.dot(x_ref[...], wr_ref[...], preferred_element_type=jnp.float32)
    logits = logits + br_ref[...]
    col = lax.broadcasted_iota(jnp.int32, logits.shape, 1)
    neg = jnp.float32(-1e30)
    lg = jnp.where(col < N_EXP, logits, neg)
    m1 = jnp.max(lg, axis=1, keepdims=True)
    i1 = jnp.min(jnp.where(lg == m1, col, 128), axis=1, keepdims=True)
    lg2 = jnp.where(col == i1, neg, lg)
    m2 = jnp.max(lg2, axis=1, keepdims=True)
    i2 = jnp.min(jnp.where(lg2 == m2, col, 128), axis=1, keepdims=True)
    e2 = jnp.exp(m2 - m1)
    w1 = 1.0 / (1.0 + e2)
    w2 = e2 * w1
    idx_ref[...] = jnp.where(col == 0, i1, jnp.where(col == 1, i2, 0))
    w_ref[...] = jnp.where(col == 0, w1, jnp.where(col == 1, w2, 0.0))


def _run_router(x, Wr, br):
    wr_pad = jnp.zeros((D_MODEL, 128), jnp.float32).at[:, :N_EXP].set(Wr)
    br_pad = jnp.zeros((1, 128), jnp.float32).at[0, :N_EXP].set(br)
    idxf, wf = pl.pallas_call(
        _router_body,
        out_shape=(
            jax.ShapeDtypeStruct((N_TOK, 128), jnp.int32),
            jax.ShapeDtypeStruct((N_TOK, 128), jnp.float32),
        ),
    )(x, wr_pad, br_pad)
    return idxf[:, :K_TOP], wf[:, :K_TOP]


def _route_metadata(idx, w):
    ef = idx.reshape(-1).astype(jnp.int32)
    wfl = w.reshape(-1)
    tok = (jnp.arange(N_TOK * K_TOP, dtype=jnp.int32) // K_TOP)
    oh = (ef[:, None] == jnp.arange(N_EXP, dtype=jnp.int32)[None, :])
    counts = jnp.sum(oh.astype(jnp.int32), axis=0)
    rank = jnp.take_along_axis(jnp.cumsum(oh.astype(jnp.int32), axis=0) - 1,
                               ef[:, None], axis=1)[:, 0]
    pad_sizes = ((counts + TB - 1) // TB) * TB
    pad_start = jnp.concatenate(
        [jnp.zeros((1,), jnp.int32), jnp.cumsum(pad_sizes)[:-1].astype(jnp.int32)])
    pos = pad_start[ef] + rank
    tok_pad = jnp.zeros((R_PAD,), jnp.int32).at[pos].set(tok)
    roww = jnp.zeros((R_PAD,), jnp.float32).at[pos].set(wfl)
    total_pad = jnp.sum(pad_sizes)
    bstart = jnp.arange(NB, dtype=jnp.int32) * TB
    bexp = jnp.clip(
        jnp.sum((bstart[:, None] >= pad_start[None, :]).astype(jnp.int32), axis=1) - 1,
        0, N_EXP - 1)
    bvalid = (bstart < total_pad).astype(jnp.int32)
    p0 = pos.reshape(N_TOK, K_TOP)[:, 0]
    p1 = pos.reshape(N_TOK, K_TOP)[:, 1]
    return tok_pad, roww, bexp, bvalid, p0, p1


_info = plsc.get_sparse_core_info()
_NC, _NS = _info.num_cores, _info.num_subcores
_NW = _NC * _NS
_GROWS = R_PAD // _NW
_GCH = 32
_TROWS = N_TOK // _NW
_CCH = 32
_sc_mesh = plsc.VectorSubcoreMesh(core_axis_name="c", subcore_axis_name="s")


@functools.partial(
    pl.kernel, mesh=_sc_mesh,
    out_type=jax.ShapeDtypeStruct((R_PAD, D_MODEL), jnp.float32),
    scratch_types=[
        pltpu.VMEM((_GROWS,), jnp.int32),
        pltpu.VMEM((_GCH, D_MODEL), jnp.float32),
        pltpu.SemaphoreType.DMA,
    ],
)
def _sc_gather_rows(x_hbm, tok_hbm, out_hbm, idx_v, rows_v, sem):
    wid = lax.axis_index("s") * _NC + lax.axis_index("c")
    base = wid * _GROWS
    pltpu.sync_copy(tok_hbm.at[pl.ds(base, _GROWS)], idx_v)
    for c in range(_GROWS // _GCH):
        pltpu.async_copy(x_hbm.at[idx_v.at[pl.ds(c * _GCH, _GCH)]], rows_v, sem).wait()
        pltpu.sync_copy(rows_v, out_hbm.at[pl.ds(base + c * _GCH, _GCH)])


@functools.partial(
    pl.kernel, mesh=_sc_mesh,
    out_type=jax.ShapeDtypeStruct((N_TOK, D_MODEL), jnp.float32),
    scratch_types=[
        pltpu.VMEM((_TROWS,), jnp.int32),
        pltpu.VMEM((_TROWS,), jnp.int32),
        pltpu.VMEM((_CCH, D_MODEL), jnp.float32),
        pltpu.VMEM((_CCH, D_MODEL), jnp.float32),
        pltpu.SemaphoreType.DMA,
        pltpu.SemaphoreType.DMA,
    ],
)
def _sc_combine(ys_hbm, p0_hbm, p1_hbm, out_hbm, i0_v, i1_v, a_v, b_v, s0, s1):
    wid = lax.axis_index("s") * _NC + lax.axis_index("c")
    base = wid * _TROWS
    pltpu.sync_copy(p0_hbm.at[pl.ds(base, _TROWS)], i0_v)
    pltpu.sync_copy(p1_hbm.at[pl.ds(base, _TROWS)], i1_v)
    nvec = D_MODEL // 16
    for c in range(_TROWS // _CCH):
        cp0 = pltpu.async_copy(ys_hbm.at[i0_v.at[pl.ds(c * _CCH, _CCH)]], a_v, s0)
        cp1 = pltpu.async_copy(ys_hbm.at[i1_v.at[pl.ds(c * _CCH, _CCH)]], b_v, s1)
        cp0.wait()
        cp1.wait()

        def body(i, carry):
            r = i // nvec
            j = (i % nvec) * 16
            a_v[r, pl.ds(j, 16)] = a_v[r, pl.ds(j, 16)] + b_v[r, pl.ds(j, 16)]
            return carry

        lax.fori_loop(0, _CCH * nvec, body, 0)
        pltpu.sync_copy(a_v, out_hbm.at[pl.ds(base + c * _CCH, _CCH)])


def _ffn_body(be_s, bv_s, xs_ref, w1_ref, b1_ref, w2_ref, b2_ref, ww_ref, out_ref):
    b = pl.program_id(0)
    h = pl.program_id(1)

    @pl.when(h == 0)
    def _():
        out_ref[...] = jnp.zeros_like(out_ref)

    valid = bv_s[b] == 1

    @pl.when(valid)
    def _():
        t = jnp.dot(xs_ref[...], w1_ref[0], preferred_element_type=jnp.float32)
        t = t + b1_ref[0]
        g = 0.5 * t * (1.0 + jnp.tanh(0.7978845608028654 * (t + 0.044715 * t * t * t)))
        out_ref[...] += jnp.dot(g, w2_ref[0], preferred_element_type=jnp.float32)

    @pl.when(valid & (h == NH - 1))
    def _():
        out_ref[...] = (out_ref[...] + b2_ref[0]) * ww_ref[:, :1]


def _run_ffn(xs, W1, b1, W2, b2, roww, bexp, bvalid):
    roww2d = jnp.tile(roww.reshape(NB * TB, 1), (1, 128))
    grid_spec = pltpu.PrefetchScalarGridSpec(
        num_scalar_prefetch=2,
        grid=(NB, NH),
        in_specs=[
            pl.BlockSpec((TB, D_MODEL), lambda b, h, be, bv: (b, 0)),
            pl.BlockSpec((1, D_MODEL, TH), lambda b, h, be, bv: (be[b], 0, h)),
            pl.BlockSpec((1, 1, TH), lambda b, h, be, bv: (be[b], 0, h)),
            pl.BlockSpec((1, TH, D_MODEL), lambda b, h, be, bv: (be[b], h, 0)),
            pl.BlockSpec((1, 1, D_MODEL), lambda b, h, be, bv: (be[b], 0, 0)),
            pl.BlockSpec((TB, 128), lambda b, h, be, bv: (b, 0)),
        ],
        out_specs=pl.BlockSpec((TB, D_MODEL), lambda b, h, be, bv: (b, 0)),
    )
    return pl.pallas_call(
        _ffn_body,
        grid_spec=grid_spec,
        out_shape=jax.ShapeDtypeStruct((R_PAD, D_MODEL), jnp.float32),
    )(bexp, bvalid, xs, W1, b1.reshape(N_EXP, 1, HIDDEN), W2,
      b2.reshape(N_EXP, 1, D_MODEL), roww2d)


def kernel(x, Wr, br, W1, b1, W2, b2):
    idx, w = _run_router(x, Wr, br)
    tok_pad, roww, bexp, bvalid, p0, p1 = _route_metadata(idx, w)
    xs = _sc_gather_rows(x, tok_pad)
    ys = _run_ffn(xs, W1, b1, W2, b2, roww, bexp, bvalid)
    return _sc_combine(ys, p0, p1)
